# baseline (speedup 1.0000x reference)
import functools

import jax
import jax.numpy as jnp
from jax import lax
from jax.experimental import pallas as pl
from jax.experimental.pallas import tpu as pltpu
from jax.experimental.pallas import tpu_sc as plsc

_NUM_WORKERS = 32
_LANES = 16
_BLK = 128


def _pick_chunk(total, cap):
    c = cap
    while c >= _LANES:
        if total % c == 0 and c % _LANES == 0:
            return c
        c -= _LANES
    raise ValueError(f"no chunk for {total}")


def _pick_cb(blocks_lo, cap):
    for cb in range(cap, 0, -1):
        if blocks_lo % cb == 0:
            return cb
    return 1


@functools.lru_cache(maxsize=None)
def _build(n_types_pad, n_atoms, n_pairs, interpret=False):
    assert n_pairs % _BLK == 0
    nb = n_pairs // _BLK
    blocks_lo = nb // _NUM_WORKERS
    n_tail = nb - blocks_lo * _NUM_WORKERS
    CB = _pick_cb(blocks_lo, 11)
    n_chunks = blocks_lo // CB
    CP = CB * _BLK
    NBUF = max(1, min(3, n_chunks))
    assert n_atoms % _LANES == 0
    n_grp = n_atoms // _LANES
    GS = -(-n_grp // 16)
    AC = GS * _LANES
    IDXSZ = max(2 * CP, AC)

    mesh = plsc.VectorSubcoreMesh(core_axis_name="c", subcore_axis_name="s")

    @functools.partial(
        pl.kernel,
        out_type=jax.ShapeDtypeStruct((n_pairs,), jnp.float32),
        mesh=mesh,
        scratch_types=(
            [
                pltpu.VMEM((n_types_pad,), jnp.float32),
                pltpu.VMEM((n_atoms,), jnp.float32),
                pltpu.VMEM((IDXSZ,), jnp.int32),
            ]
            + [pltpu.VMEM((2 * CP,), jnp.int32)
               for _ in range(NBUF - 1)]
            + [pltpu.VMEM((CP,), jnp.float32)
               for _ in range(NBUF)]
            + [pltpu.VMEM((CP,), jnp.float32)
               for _ in range(NBUF)]
            + [pltpu.VMEM_SHARED((n_atoms,), jnp.float32)]
            + [pltpu.SemaphoreType.DMA for _ in range(2 * NBUF)]
        ),
        compiler_params=pltpu.CompilerParams(
            needs_layout_passes=False, use_tc_tiling_on_sc=False
        ),
        interpret=interpret,
    )
    def vdw_kernel(vdw_hbm, anum_hbm, idx_hbm, dist_hbm, out_hbm,
                   vdw_v, rad_v, *rest):
        idxs = rest[0:NBUF]
        dists = rest[NBUF:2 * NBUF]
        outs = rest[2 * NBUF:3 * NBUF]
        rad_sh = rest[3 * NBUF]
        sem_i = rest[3 * NBUF + 1:4 * NBUF + 1]
        sem_o = rest[4 * NBUF + 1:5 * NBUF + 1]
        idx_v = idxs[0]
        sid = lax.axis_index("s")
        wid = sid * 2 + lax.axis_index("c")
        pltpu.sync_copy(vdw_hbm, vdw_v)

        a0 = lax.min(sid * GS, n_grp - GS) * _LANES
        pltpu.sync_copy(anum_hbm.at[pl.ds(a0, AC)], idx_v.at[pl.ds(0, AC)])

        @plsc.parallel_loop(0, GS, unroll=4)
        def grp(g):
            nums = idx_v[pl.ds(g * _LANES, _LANES)]
            rad = plsc.load_gather(vdw_v, [nums])
            rad_v[pl.ds(a0 + g * _LANES, _LANES)] = rad

        pltpu.sync_copy(rad_v.at[pl.ds(a0, AC)], rad_sh.at[pl.ds(a0, AC)])
        plsc.subcore_barrier()
        pltpu.sync_copy(rad_sh, rad_v)

        b0 = wid * blocks_lo + lax.min(wid, n_tail)
        bufs = tuple((idxs[k], dists[k], outs[k], sem_i[k], sem_o[k])
                     for k in range(NBUF))

        def in_copies(c, buf):
            iv, dv, _, si, _ = buf
            boff = b0 + c * CB
            return (
                pltpu.make_async_copy(
                    idx_hbm.at[pl.ds(boff * 2 * _BLK, 2 * CP)],
                    iv.at[pl.ds(0, 2 * CP)], si),
                pltpu.make_async_copy(
                    dist_hbm.at[pl.ds(boff * _BLK, CP)], dv, si),
            )

        def out_copy(c, buf):
            _, _, ov, _, so = buf
            boff = b0 + c * CB
            return pltpu.make_async_copy(
                ov, out_hbm.at[pl.ds(boff * _BLK, CP)], so)

        def start_in(c, buf):
            for cp in in_copies(c, buf):
                cp.start()

        def wait_in(c, buf):
            for cp in in_copies(c, buf):
                cp.wait()

        def do_blocks(buf, nblocks):
            iv, dv, ov, _, _ = buf
            @plsc.parallel_loop(0, nblocks, unroll=2)
            def blk(bb):
                ibase = bb * (2 * _BLK)
                pbase = bb * _BLK
                for r in range(_BLK // _LANES):
                    ii = iv[pl.ds(ibase + r * _LANES, _LANES)]
                    jj = iv[pl.ds(ibase + _BLK + r * _LANES, _LANES)]
                    ri = plsc.load_gather(rad_v, [ii])
                    rj = plsc.load_gather(rad_v, [jj])
                    d = dv[pl.ds(pbase + r * _LANES, _LANES)]
                    ov[pl.ds(pbase + r * _LANES, _LANES)] = (
                        (ri + rj) * 0.5 / d)

        def half(i, c, buf):
            wait_in(c, buf)
            @pl.when(i > 0)
            def _():
                out_copy(c, buf).wait()
            do_blocks(buf, CB)
            out_copy(c, buf).start()
            @pl.when(c + NBUF < n_chunks)
            def _():
                start_in(c + NBUF, buf)

        n_main = n_chunks // NBUF
        n_epi = n_chunks % NBUF
        for k in range(min(NBUF, n_chunks)):
            start_in(k, bufs[k])

        def ring_iter(i, _):
            for k in range(NBUF):
                half(i, i * NBUF + k, bufs[k])
            return 0
        lax.fori_loop(0, n_main, ring_iter, 0, unroll=False)

        for k in range(n_epi):
            c = n_main * NBUF + k
            wait_in(c, bufs[k])
            if c >= NBUF:
                out_copy(c, bufs[k]).wait()
            do_blocks(bufs[k], CB)
            out_copy(c, bufs[k]).start()

        for k in range(min(NBUF, n_chunks)):
            out_copy(0, bufs[k]).wait()

        @pl.when(wid < n_tail)
        def _tail():
            boff = b0 + blocks_lo
            pltpu.sync_copy(idx_hbm.at[pl.ds(boff * 2 * _BLK, 2 * _BLK)],
                            idx_v.at[pl.ds(0, 2 * _BLK)])
            pltpu.sync_copy(dist_hbm.at[pl.ds(boff * _BLK, _BLK)],
                            dists[0].at[pl.ds(0, _BLK)])
            do_blocks(bufs[0], 1)
            pltpu.sync_copy(outs[0].at[pl.ds(0, _BLK)],
                            out_hbm.at[pl.ds(boff * _BLK, _BLK)])

    return vdw_kernel


def kernel(atom_vdw, atoms_long, batch_atom_ij_idx, batch_dist_ij):
    n_types = atom_vdw.shape[0]
    n_pairs = batch_dist_ij.shape[0]
    n_types_pad = max(128, -(-n_types // 8) * 8)
    vdw_pad = jnp.zeros((n_types_pad,), jnp.float32).at[:n_types].set(atom_vdw)
    anum = atoms_long[:, 1]
    idx_flat = (
        batch_atom_ij_idx.reshape(n_pairs // _BLK, _BLK, 2)
        .transpose(0, 2, 1)
        .reshape(-1)
    )
    fn = _build(n_types_pad, atoms_long.shape[0], n_pairs)
    return fn(vdw_pad, anum, idx_flat, batch_dist_ij)


# speedup vs baseline: 1.1457x; 1.1457x over previous
import functools

import jax
import jax.numpy as jnp
from jax import lax
from jax.experimental import pallas as pl
from jax.experimental.pallas import tpu as pltpu
from jax.experimental.pallas import tpu_sc as plsc

_NUM_WORKERS = 32
_LANES = 16
_BLK = 128


def _pick_chunk(total, cap):
    c = cap
    while c >= _LANES:
        if total % c == 0 and c % _LANES == 0:
            return c
        c -= _LANES
    raise ValueError(f"no chunk for {total}")


def _pick_cb(blocks_lo, cap):
    for cb in range(cap, 0, -1):
        if blocks_lo % cb == 0:
            return cb
    return 1


@functools.lru_cache(maxsize=None)
def _build(n_types_pad, n_atoms, n_pairs, interpret=False):
    assert n_pairs % _BLK == 0
    nb = n_pairs // _BLK
    blocks_lo = nb // _NUM_WORKERS
    n_tail = nb - blocks_lo * _NUM_WORKERS
    CB = min(16, blocks_lo)
    n_chunks = blocks_lo // CB
    rem_blocks = blocks_lo - n_chunks * CB
    CP = CB * _BLK
    NBUF = max(1, min(3, n_chunks))
    assert n_atoms % _LANES == 0
    n_grp = n_atoms // _LANES
    GS = -(-n_grp // 16)
    AC = GS * _LANES
    p1_chunks = []
    o = 0
    while o < AC:
        s = min(2 * CP, AC - o)
        p1_chunks.append((o, s))
        o += s

    mesh = plsc.VectorSubcoreMesh(core_axis_name="c", subcore_axis_name="s")

    @functools.partial(
        pl.kernel,
        out_type=jax.ShapeDtypeStruct((n_pairs,), jnp.float32),
        mesh=mesh,
        scratch_types=(
            [
                pltpu.VMEM((n_types_pad,), jnp.float32),
                pltpu.VMEM((n_atoms,), jnp.float32),
            ]
            + [pltpu.VMEM((2 * CP,), jnp.int32)
               for _ in range(NBUF)]
            + [pltpu.VMEM((CP,), jnp.float32)
               for _ in range(NBUF)]
            + [pltpu.VMEM((CP,), jnp.float32)
               for _ in range(NBUF)]
            + [pltpu.VMEM_SHARED((n_atoms,), jnp.float32)]
            + [pltpu.SemaphoreType.DMA for _ in range(2 * NBUF)]
        ),
        compiler_params=pltpu.CompilerParams(
            needs_layout_passes=False, use_tc_tiling_on_sc=False
        ),
        interpret=interpret,
    )
    def vdw_kernel(vdw_hbm, anum_hbm, idx_hbm, dist_hbm, out_hbm,
                   vdw_v, rad_v, *rest):
        idxs = rest[0:NBUF]
        dists = rest[NBUF:2 * NBUF]
        outs = rest[2 * NBUF:3 * NBUF]
        rad_sh = rest[3 * NBUF]
        sem_i = rest[3 * NBUF + 1:4 * NBUF + 1]
        sem_o = rest[4 * NBUF + 1:5 * NBUF + 1]
        idx_v = idxs[0]
        sid = lax.axis_index("s")
        wid = sid * 2 + lax.axis_index("c")
        pltpu.sync_copy(vdw_hbm, vdw_v)

        a0 = lax.min(sid * GS, n_grp - GS) * _LANES
        for p1_off, p1_sz in p1_chunks:
            pltpu.sync_copy(anum_hbm.at[pl.ds(a0 + p1_off, p1_sz)],
                            idx_v.at[pl.ds(0, p1_sz)])

            @plsc.parallel_loop(0, p1_sz // _LANES, unroll=4)
            def grp(g, _o=p1_off):
                nums = idx_v[pl.ds(g * _LANES, _LANES)]
                rad = plsc.load_gather(vdw_v, [nums])
                rad_v[pl.ds(a0 + _o + g * _LANES, _LANES)] = rad

        pltpu.sync_copy(rad_v.at[pl.ds(a0, AC)], rad_sh.at[pl.ds(a0, AC)])
        plsc.subcore_barrier()
        pltpu.sync_copy(rad_sh, rad_v)

        b0 = wid * blocks_lo + lax.min(wid, n_tail)
        bufs = tuple((idxs[k], dists[k], outs[k], sem_i[k], sem_o[k])
                     for k in range(NBUF))

        def in_copies(c, buf):
            iv, dv, _, si, _ = buf
            boff = b0 + c * CB
            return (
                pltpu.make_async_copy(
                    idx_hbm.at[pl.ds(boff * 2 * _BLK, 2 * CP)],
                    iv.at[pl.ds(0, 2 * CP)], si),
                pltpu.make_async_copy(
                    dist_hbm.at[pl.ds(boff * _BLK, CP)], dv, si),
            )

        def out_copy(c, buf):
            _, _, ov, _, so = buf
            boff = b0 + c * CB
            return pltpu.make_async_copy(
                ov, out_hbm.at[pl.ds(boff * _BLK, CP)], so)

        def start_in(c, buf):
            for cp in in_copies(c, buf):
                cp.start()

        def wait_in(c, buf):
            for cp in in_copies(c, buf):
                cp.wait()

        def do_blocks(buf, nblocks):
            iv, dv, ov, _, _ = buf
            @plsc.parallel_loop(0, nblocks, unroll=2)
            def blk(bb):
                ibase = bb * (2 * _BLK)
                pbase = bb * _BLK
                for r in range(_BLK // _LANES):
                    ii = iv[pl.ds(ibase + r * _LANES, _LANES)]
                    jj = iv[pl.ds(ibase + _BLK + r * _LANES, _LANES)]
                    ri = plsc.load_gather(rad_v, [ii])
                    rj = plsc.load_gather(rad_v, [jj])
                    d = dv[pl.ds(pbase + r * _LANES, _LANES)]
                    ov[pl.ds(pbase + r * _LANES, _LANES)] = (
                        (ri + rj) * 0.5 / d)

        def half(i, c, buf):
            wait_in(c, buf)
            @pl.when(i > 0)
            def _():
                out_copy(c, buf).wait()
            do_blocks(buf, CB)
            @pl.when(c + NBUF < n_chunks)
            def _():
                start_in(c + NBUF, buf)
            out_copy(c, buf).start()

        n_main = n_chunks // NBUF
        n_epi = n_chunks % NBUF
        for k in range(min(NBUF, n_chunks)):
            start_in(k, bufs[k])

        def ring_iter(i, _):
            for k in range(NBUF):
                half(i, i * NBUF + k, bufs[k])
            return 0
        lax.fori_loop(0, n_main, ring_iter, 0, unroll=False)

        for k in range(n_epi):
            c = n_main * NBUF + k
            wait_in(c, bufs[k])
            if c >= NBUF:
                out_copy(c, bufs[k]).wait()
            do_blocks(bufs[k], CB)
            out_copy(c, bufs[k]).start()

        for k in range(min(NBUF, n_chunks)):
            out_copy(0, bufs[k]).wait()

        if rem_blocks:
            boff_r = b0 + n_chunks * CB
            pltpu.sync_copy(
                idx_hbm.at[pl.ds(boff_r * 2 * _BLK, rem_blocks * 2 * _BLK)],
                idx_v.at[pl.ds(0, rem_blocks * 2 * _BLK)])
            pltpu.sync_copy(
                dist_hbm.at[pl.ds(boff_r * _BLK, rem_blocks * _BLK)],
                dists[0].at[pl.ds(0, rem_blocks * _BLK)])
            do_blocks(bufs[0], rem_blocks)
            pltpu.sync_copy(
                outs[0].at[pl.ds(0, rem_blocks * _BLK)],
                out_hbm.at[pl.ds(boff_r * _BLK, rem_blocks * _BLK)])

        @pl.when(wid < n_tail)
        def _tail():
            boff = b0 + blocks_lo
            pltpu.sync_copy(idx_hbm.at[pl.ds(boff * 2 * _BLK, 2 * _BLK)],
                            idx_v.at[pl.ds(0, 2 * _BLK)])
            pltpu.sync_copy(dist_hbm.at[pl.ds(boff * _BLK, _BLK)],
                            dists[0].at[pl.ds(0, _BLK)])
            do_blocks(bufs[0], 1)
            pltpu.sync_copy(outs[0].at[pl.ds(0, _BLK)],
                            out_hbm.at[pl.ds(boff * _BLK, _BLK)])

    return vdw_kernel


def kernel(atom_vdw, atoms_long, batch_atom_ij_idx, batch_dist_ij):
    n_types = atom_vdw.shape[0]
    n_pairs = batch_dist_ij.shape[0]
    n_types_pad = max(128, -(-n_types // 8) * 8)
    vdw_pad = jnp.zeros((n_types_pad,), jnp.float32).at[:n_types].set(atom_vdw)
    anum = atoms_long[:, 1]
    idx_flat = (
        batch_atom_ij_idx.reshape(n_pairs // _BLK, _BLK, 2)
        .transpose(0, 2, 1)
        .reshape(-1)
    )
    fn = _build(n_types_pad, atoms_long.shape[0], n_pairs)
    return fn(vdw_pad, anum, idx_flat, batch_dist_ij)
